# baseline (speedup 1.0000x reference)
import functools

import jax
import jax.numpy as jnp
from jax.experimental import pallas as pl
from jax.experimental.pallas import tpu as pltpu

_N = 10000
_M = 10000
_POWER = 2.0


def _shift_normalize(H, node, edge, m, power):
    Hc = jnp.clip(H, 1e-07, 10.0)
    Hp = Hc ** power
    ones = jnp.ones(node.shape[0], dtype=jnp.float32)
    Spow = jax.ops.segment_sum(Hp[node], edge, num_segments=_M)
    deg = jax.ops.segment_sum(ones, edge, num_segments=_M)
    neigh = deg[edge] - m
    denom = jnp.maximum(deg[edge] - 1.0, 1.0)
    num = Spow[edge] - Hp[node] * m[:, None]
    mask = (neigh[:, None] > 0.0) & (num > 0.0)
    safe = jnp.where(mask, num, 1.0)
    contrib = jnp.where(mask, (safe / denom[:, None]) ** (1.0 / power), 0.0)
    new_signal = Hc + jnp.zeros_like(Hc).at[node].add(contrib)
    rowsum = new_signal.sum(1)
    r_inv = jnp.where(rowsum > 0.0, 1.0 / jnp.maximum(rowsum, 1e-30), 0.0)
    return new_signal * r_inv[:, None]


def _dense_body(h_ref, w_ref, b_ref, o_ref):
    o_ref[...] = h_ref[...] @ w_ref[...] + b_ref[...]


def _multiplicity(node, edge):
    pair = edge.astype(jnp.int64) * _N + node.astype(jnp.int64)
    order = jnp.argsort(pair)
    ps = pair[order]
    starts = jnp.concatenate(
        [jnp.ones((1,), dtype=jnp.int32), (ps[1:] != ps[:-1]).astype(jnp.int32)])
    gid = jnp.cumsum(starts) - 1
    ones = jnp.ones(node.shape[0], dtype=jnp.float32)
    gsize = jax.ops.segment_sum(ones, gid, num_segments=node.shape[0])
    m_sorted = gsize[gid]
    return jnp.zeros_like(m_sorted).at[order].set(m_sorted)


def kernel(x, edge_index, W1, b1, W2, b2):
    node = edge_index[0]
    edge = edge_index[1]
    m = _multiplicity(node, edge)
    H = _shift_normalize(x, node, edge, m, _POWER)
    H = pl.pallas_call(
        _dense_body,
        out_shape=jax.ShapeDtypeStruct((_N, W1.shape[1]), jnp.float32),
    )(H, W1, b1)
    H = jax.nn.relu(H)
    H = _shift_normalize(H, node, edge, m, _POWER)
    H = pl.pallas_call(
        _dense_body,
        out_shape=jax.ShapeDtypeStruct((_N, W2.shape[1]), jnp.float32),
    )(H, W2, b2)
    return H


# speedup vs baseline: 2.1046x; 2.1046x over previous
import functools

import jax
import jax.numpy as jnp
from jax import lax
from jax.experimental import pallas as pl
from jax.experimental.pallas import tpu as pltpu
from jax.experimental.pallas import tpu_sc as plsc

_N = 10000
_M = 10000
_E = 320000
_D = 128
_HID = 16
_C = 32
_NC = 2
_NS = 16
_L = 16
_B = 80

_MESH = dict(core_axis_name="c", subcore_axis_name="s", num_cores=_NC,
             num_subcores=_NS)


def _splat(v, dtype=jnp.int32):
    return jnp.full((_L,), v, dtype=dtype)


def _fill16(ref, off, val, dtype=jnp.float32):
    ref[pl.ds(off, _L)] = jnp.full((_L,), val, dtype=dtype)


def _rsqrt16(x):
    bi = plsc.bitcast(x, jnp.int32)
    y = plsc.bitcast(jnp.int32(0x5F3759DF) - (bi >> 1), jnp.float32)
    y = y * (1.5 - 0.5 * x * y * y)
    y = y * (1.5 - 0.5 * x * y * y)
    return y


def _sqrt16(x):
    return x * _rsqrt16(x)


def _row_split(s, fn640, fn400):
    @pl.when(s < _NS - 1)
    def _():
        fn640(s * 640, 640)

    @pl.when(s == _NS - 1)
    def _():
        fn400(9600, 400)


def _stats_body(node_hbm, edge_hbm, t_hbm, mflt_hbm, degf_hbm,
                nidx, eidx, pbuf, vbuf, wbuf, fmb, onesb, zbuf, degS, cS):
    c = lax.axis_index("c")
    s = lax.axis_index("s")
    ept = _E // _NS
    nb = ept // _B

    for k in range(_B // _L):
        _fill16(onesb, k * _L, 1.0)
    def _zb(i, _):
        _fill16(zbuf, i * _L, 0.0)
        return 0
    lax.fori_loop(0, 2000 // _L, _zb, 0)

    _row_split(s, lambda r0, n: pltpu.sync_copy(zbuf.at[pl.ds(0, 640)],
                                                degS.at[pl.ds(r0, 640)]),
               lambda r0, n: pltpu.sync_copy(zbuf.at[pl.ds(0, 400)],
                                             degS.at[pl.ds(r0, 400)]))
    def _zc(i, _):
        pltpu.sync_copy(zbuf.at[pl.ds(0, 2000)],
                        cS.at[pl.ds(s * ept + i * 2000, 2000)])
        return 0
    lax.fori_loop(0, ept // 2000, _zc, 0)
    plsc.subcore_barrier()

    def _load_pair(g):
        b0 = s * ept + g * _B
        pltpu.sync_copy(node_hbm.at[pl.ds(b0, _B)], nidx)
        pltpu.sync_copy(edge_hbm.at[pl.ds(b0, _B)], eidx)
        for k in range(_B // _L):
            ev = eidx[pl.ds(k * _L, _L)]
            nv = nidx[pl.ds(k * _L, _L)]
            pbuf[pl.ds(k * _L, _L)] = ev * _N + nv
        return b0

    @pl.when(c == 0)
    def _p1():
        def _g(g, _):
            b0 = _load_pair(g)
            for k in range(_B // _L):
                vbuf[pl.ds(k * _L, _L)] = (_splat(b0 + k * _L)
                                           + lax.iota(jnp.int32, _L))
            pltpu.sync_copy(vbuf, t_hbm.at[pbuf])
            pltpu.sync_copy(onesb, degS.at[eidx], add=True)
            return 0
        lax.fori_loop(0, nb, _g, 0)
    plsc.subcore_barrier()

    @pl.when(c == 0)
    def _p3():
        def _g(g, _):
            _load_pair(g)
            pltpu.sync_copy(t_hbm.at[pbuf], wbuf)
            pltpu.sync_copy(onesb, cS.at[wbuf], add=True)
            return 0
        lax.fori_loop(0, nb, _g, 0)
    plsc.subcore_barrier()

    @pl.when(c == 0)
    def _p4():
        def _g(g, _):
            b0 = _load_pair(g)
            pltpu.sync_copy(t_hbm.at[pbuf], wbuf)
            pltpu.sync_copy(cS.at[wbuf], fmb)
            pltpu.sync_copy(fmb, mflt_hbm.at[pl.ds(b0, _B)])
            return 0
        lax.fori_loop(0, nb, _g, 0)

        def _wb(r0, n):
            pltpu.sync_copy(degS.at[pl.ds(r0, n)], zbuf.at[pl.ds(0, n)])
            pltpu.sync_copy(zbuf.at[pl.ds(0, n)], degf_hbm.at[pl.ds(r0, n)])
        _row_split(s, lambda r0, n: _wb(r0, 640), lambda r0, n: _wb(r0, 400))


_stats = pl.kernel(
    _stats_body,
    out_type=[jax.ShapeDtypeStruct((_N * _M,), jnp.int32),
              jax.ShapeDtypeStruct((_E,), jnp.float32),
              jax.ShapeDtypeStruct((_M,), jnp.float32)],
    mesh=plsc.VectorSubcoreMesh(**_MESH),
    compiler_params=pltpu.CompilerParams(needs_layout_passes=False, use_tc_tiling_on_sc=False, internal_scratch_in_bytes=131072),
    scratch_types=[pltpu.VMEM((_B,), jnp.int32),
                   pltpu.VMEM((_B,), jnp.int32),
                   pltpu.VMEM((_B,), jnp.int32),
                   pltpu.VMEM((_B,), jnp.int32),
                   pltpu.VMEM((_B,), jnp.int32),
                   pltpu.VMEM((_B,), jnp.float32),
                   pltpu.VMEM((_B,), jnp.float32),
                   pltpu.VMEM((2000,), jnp.float32),
                   pltpu.VMEM_SHARED((_M,), jnp.float32),
                   pltpu.VMEM_SHARED((_E,), jnp.float32)],
)


def _staged_copy(s, sbuf, mk_hbm_ref, spm, spm_to_hbm):
    def _chunk(r0, n):
        if spm_to_hbm:
            pltpu.sync_copy(spm.at[pl.ds(r0, n)], sbuf.at[pl.ds(0, n)])
            pltpu.sync_copy(sbuf.at[pl.ds(0, n)], mk_hbm_ref(r0, n))
        else:
            pltpu.sync_copy(mk_hbm_ref(r0, n), sbuf.at[pl.ds(0, n)])
            pltpu.sync_copy(sbuf.at[pl.ds(0, n)], spm.at[pl.ds(r0, n)])

    def _640(r0, _):
        for q in range(4):
            _chunk(r0 + q * 160, 160)

    def _400(r0, _):
        for q in range(2):
            _chunk(r0 + q * 160, 160)
        _chunk(r0 + 320, 80)

    _row_split(s, _640, _400)


def _zero_shared(s, zbuf, table, rows_f):
    def _z640(r0, n):
        for q in range(4):
            pltpu.sync_copy(zbuf, table.at[pl.ds(r0 + q * 160, 160)])

    def _z400(r0, n):
        for q in range(2):
            pltpu.sync_copy(zbuf, table.at[pl.ds(r0 + q * 160, 160)])
        pltpu.sync_copy(zbuf.at[pl.ds(0, 80)], table.at[pl.ds(r0 + 320, 80)])

    _row_split(s, _z640, _z400)


def _contrib_batch(j_hi, F, nidx, eidx, mb, degb, isdb, degS_t,
                   hrows, srows, crows):
    for k in range(_B // _L):
        ev = eidx[pl.ds(k * _L, _L)]
        dv = plsc.load_gather(degS_t, [ev])
        degb[pl.ds(k * _L, _L)] = dv
        isdb[pl.ds(k * _L, _L)] = _rsqrt16(jnp.maximum(dv - 1.0, 1.0))

    def _j(j, _):
        jv = _splat(j)
        mj = plsc.load_gather(mb, [jv])
        dj = plsc.load_gather(degb, [jv])
        ij = plsc.load_gather(isdb, [jv])
        mask0 = (dj - mj) > 0.0
        for k in range(F // _L):
            sl = pl.ds(k * _L, _L)
            hp = hrows[j, sl]
            sp = srows[j, sl]
            num = sp - hp * mj
            msk = mask0 & (num > 0.0)
            safe = jnp.where(msk, num, 1.0)
            crows[j, sl] = jnp.where(msk, _sqrt16(safe) * ij, 0.0)
        return 0
    lax.fori_loop(0, j_hi, _j, 0)


def _agg1_body(hp_hbm, node_hbm, edge_hbm, mflt_hbm, degf_hbm,
               out_hbm, nidx, eidx, mb, degb, isdb, hrows, srows, crows,
               degS_t, zbuf, SpowS, OutS):
    F = _D // _NC
    c = lax.axis_index("c")
    s = lax.axis_index("s")
    ept = _E // _NS
    nb = ept // _B

    def _zb(r, _):
        for k in range(F // _L):
            zbuf[r, pl.ds(k * _L, _L)] = jnp.zeros((_L,), jnp.float32)
        return 0
    lax.fori_loop(0, 160, _zb, 0)
    _zero_shared(s, zbuf, SpowS, F)
    _zero_shared(s, zbuf, OutS, F)
    pltpu.sync_copy(degf_hbm, degS_t)
    plsc.subcore_barrier()

    def _gb(g, _):
        b0 = s * ept + g * _B
        pltpu.sync_copy(node_hbm.at[pl.ds(b0, _B)], nidx)
        pltpu.sync_copy(edge_hbm.at[pl.ds(b0, _B)], eidx)
        pltpu.sync_copy(hp_hbm.at[c].at[nidx], hrows)
        pltpu.sync_copy(hrows, SpowS.at[eidx], add=True)
        return 0
    lax.fori_loop(0, nb, _gb, 0)
    plsc.subcore_barrier()

    def _gc(g, _):
        b0 = s * ept + g * _B
        pltpu.sync_copy(node_hbm.at[pl.ds(b0, _B)], nidx)
        pltpu.sync_copy(edge_hbm.at[pl.ds(b0, _B)], eidx)
        pltpu.sync_copy(mflt_hbm.at[pl.ds(b0, _B)], mb)
        pltpu.sync_copy(hp_hbm.at[c].at[nidx], hrows)
        pltpu.sync_copy(SpowS.at[eidx], srows)
        _contrib_batch(_B, F, nidx, eidx, mb, degb, isdb, degS_t,
                       hrows, srows, crows)
        pltpu.sync_copy(crows, OutS.at[nidx], add=True)
        return 0
    lax.fori_loop(0, nb, _gc, 0)
    plsc.subcore_barrier()

    _staged_copy(s, zbuf, lambda r0, n: out_hbm.at[c, pl.ds(r0, n)], OutS,
                 spm_to_hbm=True)


_agg1 = pl.kernel(
    _agg1_body,
    out_type=jax.ShapeDtypeStruct((_NC, _N, _D // _NC), jnp.float32),
    mesh=plsc.VectorSubcoreMesh(**_MESH),
    compiler_params=pltpu.CompilerParams(needs_layout_passes=False, use_tc_tiling_on_sc=False, internal_scratch_in_bytes=131072),
    scratch_types=[pltpu.VMEM((_B,), jnp.int32),
                   pltpu.VMEM((_B,), jnp.int32),
                   pltpu.VMEM((_B,), jnp.float32),
                   pltpu.VMEM((_B,), jnp.float32),
                   pltpu.VMEM((_B,), jnp.float32),
                   pltpu.VMEM((_B, _D // _NC), jnp.float32),
                   pltpu.VMEM((_B, _D // _NC), jnp.float32),
                   pltpu.VMEM((_B, _D // _NC), jnp.float32),
                   pltpu.VMEM((_M,), jnp.float32),
                   pltpu.VMEM((160, _D // _NC), jnp.float32),
                   pltpu.VMEM_SHARED((_M, _D // _NC), jnp.float32),
                   pltpu.VMEM_SHARED((_N, _D // _NC), jnp.float32)],
)


def _agg2_body(hp_hbm, node_hbm, edge_hbm, mflt_hbm, degf_hbm,
               out_hbm, nidx, eidx, mb, degb, isdb, hrows, srows, crows,
               degS_t, zbuf, HpS, SpowS, OutS):
    F = _HID
    c = lax.axis_index("c")
    s = lax.axis_index("s")

    def _zb(r, _):
        zbuf[r, pl.ds(0, _L)] = jnp.zeros((_L,), jnp.float32)
        return 0
    _staged_copy(s, zbuf, lambda r0, n: hp_hbm.at[pl.ds(r0, n)], HpS,
                 spm_to_hbm=False)
    lax.fori_loop(0, 160, _zb, 0)
    _zero_shared(s, zbuf, SpowS, F)
    _zero_shared(s, zbuf, OutS, F)
    pltpu.sync_copy(degf_hbm, degS_t)
    plsc.subcore_barrier()

    ept_b = _E // _NS
    def _gb(g, _):
        b0 = s * ept_b + g * _B
        pltpu.sync_copy(node_hbm.at[pl.ds(b0, _B)], nidx)
        pltpu.sync_copy(edge_hbm.at[pl.ds(b0, _B)], eidx)
        pltpu.sync_copy(HpS.at[nidx], hrows)
        pltpu.sync_copy(hrows, SpowS.at[eidx], add=True)
        return 0
    lax.fori_loop(0, ept_b // _B, _gb, 0)
    plsc.subcore_barrier()

    ept_c = _E // (_NC * _NS)
    def _gc(g, _):
        b0 = (s * _NC + c) * ept_c + g * _B
        pltpu.sync_copy(node_hbm.at[pl.ds(b0, _B)], nidx)
        pltpu.sync_copy(edge_hbm.at[pl.ds(b0, _B)], eidx)
        pltpu.sync_copy(mflt_hbm.at[pl.ds(b0, _B)], mb)
        pltpu.sync_copy(HpS.at[nidx], hrows)
        pltpu.sync_copy(SpowS.at[eidx], srows)
        _contrib_batch(_B, F, nidx, eidx, mb, degb, isdb, degS_t,
                       hrows, srows, crows)
        pltpu.sync_copy(crows, OutS.at[nidx], add=True)
        return 0
    lax.fori_loop(0, ept_c // _B, _gc, 0)
    plsc.subcore_barrier()

    _staged_copy(s, zbuf, lambda r0, n: out_hbm.at[c, pl.ds(r0, n)], OutS,
                 spm_to_hbm=True)


_agg2 = pl.kernel(
    _agg2_body,
    out_type=jax.ShapeDtypeStruct((_NC, _N, _HID), jnp.float32),
    mesh=plsc.VectorSubcoreMesh(**_MESH),
    compiler_params=pltpu.CompilerParams(needs_layout_passes=False, use_tc_tiling_on_sc=False, internal_scratch_in_bytes=131072),
    scratch_types=[pltpu.VMEM((_B,), jnp.int32),
                   pltpu.VMEM((_B,), jnp.int32),
                   pltpu.VMEM((_B,), jnp.float32),
                   pltpu.VMEM((_B,), jnp.float32),
                   pltpu.VMEM((_B,), jnp.float32),
                   pltpu.VMEM((_B, _HID), jnp.float32),
                   pltpu.VMEM((_B, _HID), jnp.float32),
                   pltpu.VMEM((_B, _HID), jnp.float32),
                   pltpu.VMEM((_M,), jnp.float32),
                   pltpu.VMEM((160, _HID), jnp.float32),
                   pltpu.VMEM_SHARED((_N, _HID), jnp.float32),
                   pltpu.VMEM_SHARED((_M, _HID), jnp.float32),
                   pltpu.VMEM_SHARED((_N, _HID), jnp.float32)],
)


def _prep1_body(x_ref, hc_ref, hp_ref):
    hc = jnp.clip(x_ref[...], 1e-07, 10.0)
    hc_ref[...] = hc
    hp = hc * hc
    hp_ref[0] = hp[:, :_D // _NC]
    hp_ref[1] = hp[:, _D // _NC:]


def _post1_body(hc_ref, os_ref, w_ref, b_ref, hc2_ref, hp2_ref):
    contrib = jnp.concatenate([os_ref[0], os_ref[1]], axis=1)
    ns = hc_ref[...] + contrib
    rowsum = ns.sum(axis=1)
    rinv = jnp.where(rowsum > 0.0, 1.0 / jnp.maximum(rowsum, 1e-30), 0.0)
    h = ns * rinv[:, None]
    h = jnp.dot(h, w_ref[...], preferred_element_type=jnp.float32) + b_ref[...]
    h = jnp.maximum(h, 0.0)
    hc2 = jnp.clip(h, 1e-07, 10.0)
    hc2_ref[...] = hc2
    hp2_ref[...] = hc2 * hc2


def _post2_body(hc_ref, os_ref, w_ref, b_ref, out_ref):
    ns = hc_ref[...] + os_ref[0] + os_ref[1]
    rowsum = ns.sum(axis=1)
    rinv = jnp.where(rowsum > 0.0, 1.0 / jnp.maximum(rowsum, 1e-30), 0.0)
    h = ns * rinv[:, None]
    out_ref[...] = (jnp.dot(h, w_ref[...], preferred_element_type=jnp.float32)
                    + b_ref[...])


def kernel(x, edge_index, W1, b1, W2, b2):
    node = edge_index[0].astype(jnp.int32)
    edge = edge_index[1].astype(jnp.int32)

    _t, mflt, degf = _stats(node, edge)

    hc1, hp1 = pl.pallas_call(
        _prep1_body,
        out_shape=[jax.ShapeDtypeStruct((_N, _D), jnp.float32),
                   jax.ShapeDtypeStruct((_NC, _N, _D // _NC), jnp.float32)],
    )(x)

    os1 = _agg1(hp1, node, edge, mflt, degf)

    hc2, hp2 = pl.pallas_call(
        _post1_body,
        out_shape=[jax.ShapeDtypeStruct((_N, _HID), jnp.float32),
                   jax.ShapeDtypeStruct((_N, _HID), jnp.float32)],
    )(hc1, os1, W1, b1)

    os2 = _agg2(hp2, node, edge, mflt, degf)

    out = pl.pallas_call(
        _post2_body,
        out_shape=jax.ShapeDtypeStruct((_N, _C), jnp.float32),
    )(hc2, os2, W2, b2)
    return out


# speedup vs baseline: 3.0148x; 1.4325x over previous
import jax
import jax.numpy as jnp
from jax import lax
from jax.experimental import pallas as pl
from jax.experimental.pallas import tpu as pltpu
from jax.experimental.pallas import tpu_sc as plsc

_N = 10000
_M = 10000
_E = 320000
_D = 128
_HID = 16
_C = 32
_NC = 2
_NS = 16
_L = 16
_B = 80
_K = 5

_MESH = dict(core_axis_name="c", subcore_axis_name="s", num_cores=_NC,
             num_subcores=_NS)
_PARAMS = pltpu.CompilerParams(needs_layout_passes=False,
                               use_tc_tiling_on_sc=False)


def _splat(v, dtype=jnp.int32):
    return jnp.full((_L,), v, dtype=dtype)


def _fill16(ref, off, val, dtype=jnp.float32):
    ref[pl.ds(off, _L)] = jnp.full((_L,), val, dtype=dtype)


def _rsqrt16(x):
    bi = plsc.bitcast(x, jnp.int32)
    y = plsc.bitcast(jnp.int32(0x5F3759DF) - (bi >> 1), jnp.float32)
    y = y * (1.5 - 0.5 * x * y * y)
    y = y * (1.5 - 0.5 * x * y * y)
    return y


def _sqrt16(x):
    return x * _rsqrt16(x)


def _row_split(s, fn640, fn400):
    @pl.when(s < _NS - 1)
    def _():
        fn640(s * 640, 640)

    @pl.when(s == _NS - 1)
    def _():
        fn400(9600, 400)


def _drain(ds):
    for d in ds:
        d.wait()


def _stats_body(node_hbm, edge_hbm, t_hbm, mflt_hbm, degf_hbm, *refs):
    nbufs = refs[0:_K]
    ebufs = refs[_K:2 * _K]
    pbufs = refs[2 * _K:3 * _K]
    vbufs = refs[3 * _K:4 * _K]
    wbufs = refs[4 * _K:5 * _K]
    fmbs = refs[5 * _K:6 * _K]
    onesb, zbuf, sem_i, sem_g, sem_w, sem_a, degS, cS = refs[6 * _K:]

    c = lax.axis_index("c")
    s = lax.axis_index("s")
    ept = _E // _NS
    nchunk = ept // (_B * _K)

    for k in range(_B // _L):
        _fill16(onesb, k * _L, 1.0)
    def _zb(i, _):
        _fill16(zbuf, i * _L, 0.0)
        return 0
    lax.fori_loop(0, 2000 // _L, _zb, 0)

    _row_split(s, lambda r0, n: pltpu.sync_copy(zbuf.at[pl.ds(0, 640)],
                                                degS.at[pl.ds(r0, 640)]),
               lambda r0, n: pltpu.sync_copy(zbuf.at[pl.ds(0, 400)],
                                             degS.at[pl.ds(r0, 400)]))
    def _zc(i, _):
        pltpu.sync_copy(zbuf.at[pl.ds(0, 2000)],
                        cS.at[pl.ds(s * ept + i * 2000, 2000)])
        return 0
    lax.fori_loop(0, ept // 2000, _zc, 0)
    plsc.subcore_barrier()

    def _fire_idx(ci):
        for b in range(_K):
            b0 = s * ept + (ci * _K + b) * _B
            pltpu.sync_copy(node_hbm.at[pl.ds(b0, _B)], nbufs[b])
            pltpu.sync_copy(edge_hbm.at[pl.ds(b0, _B)], ebufs[b])
        return []

    def _mk_pairs():
        for b in range(_K):
            for k in range(_B // _L):
                sl = pl.ds(k * _L, _L)
                pbufs[b][sl] = ebufs[b][sl] * _N + nbufs[b][sl]

    @pl.when(c == 0)
    def _p1():
        def _chunk(ci, _):
            _drain(_fire_idx(ci))
            _mk_pairs()
            ds, da = [], []
            for b in range(_K):
                b0 = s * ept + (ci * _K + b) * _B
                for k in range(_B // _L):
                    vbufs[b][pl.ds(k * _L, _L)] = (_splat(b0 + k * _L)
                                                   + lax.iota(jnp.int32, _L))
                pltpu.sync_copy(vbufs[b], t_hbm.at[pbufs[b]])
                pltpu.sync_copy(onesb, degS.at[ebufs[b]], add=True)
            _drain(ds)
            return 0
        lax.fori_loop(0, nchunk, _chunk, 0)
    plsc.subcore_barrier()

    @pl.when(c == 0)
    def _p2():
        def _chunk(ci, _):
            _drain(_fire_idx(ci))
            _mk_pairs()
            for b in range(_K):
                pltpu.sync_copy(t_hbm.at[pbufs[b]], wbufs[b])
                pltpu.sync_copy(onesb, cS.at[wbufs[b]], add=True)
            return 0
        lax.fori_loop(0, nchunk, _chunk, 0)
    plsc.subcore_barrier()

    @pl.when(c == 0)
    def _p3():
        def _chunk(ci, _):
            _drain(_fire_idx(ci))
            _mk_pairs()
            for b in range(_K):
                b0 = s * ept + (ci * _K + b) * _B
                pltpu.sync_copy(t_hbm.at[pbufs[b]], wbufs[b])
                pltpu.sync_copy(cS.at[wbufs[b]], fmbs[b])
                pltpu.sync_copy(fmbs[b], mflt_hbm.at[pl.ds(b0, _B)])
            return 0
        lax.fori_loop(0, nchunk, _chunk, 0)

        def _wb(r0, n):
            pltpu.sync_copy(degS.at[pl.ds(r0, n)], zbuf.at[pl.ds(0, n)])
            pltpu.sync_copy(zbuf.at[pl.ds(0, n)], degf_hbm.at[pl.ds(r0, n)])
        _row_split(s, lambda r0, n: _wb(r0, 640), lambda r0, n: _wb(r0, 400))


_stats = pl.kernel(
    _stats_body,
    out_type=[jax.ShapeDtypeStruct((_N * _M,), jnp.int32),
              jax.ShapeDtypeStruct((_E,), jnp.float32),
              jax.ShapeDtypeStruct((_M,), jnp.float32)],
    mesh=plsc.VectorSubcoreMesh(**_MESH),
    compiler_params=_PARAMS,
    scratch_types=([pltpu.VMEM((_B,), jnp.int32)] * (5 * _K)
                   + [pltpu.VMEM((_B,), jnp.float32)] * _K
                   + [pltpu.VMEM((_B,), jnp.float32),
                      pltpu.VMEM((2000,), jnp.float32),
                      pltpu.SemaphoreType.DMA,
                      pltpu.SemaphoreType.DMA,
                      pltpu.SemaphoreType.DMA,
                      pltpu.SemaphoreType.DMA,
                      pltpu.VMEM_SHARED((_M,), jnp.float32),
                      pltpu.VMEM_SHARED((_E,), jnp.float32)]),
)


def _staged_copy(s, sbuf, mk_hbm_ref, spm, spm_to_hbm):
    def _chunk(r0, n):
        if spm_to_hbm:
            pltpu.sync_copy(spm.at[pl.ds(r0, n)], sbuf.at[pl.ds(0, n)])
            pltpu.sync_copy(sbuf.at[pl.ds(0, n)], mk_hbm_ref(r0, n))
        else:
            pltpu.sync_copy(mk_hbm_ref(r0, n), sbuf.at[pl.ds(0, n)])
            pltpu.sync_copy(sbuf.at[pl.ds(0, n)], spm.at[pl.ds(r0, n)])

    def _640(r0, _):
        for q in range(8):
            _chunk(r0 + q * 80, 80)

    def _400(r0, _):
        for q in range(5):
            _chunk(r0 + q * 80, 80)

    _row_split(s, _640, _400)


def _zero_shared(s, zbuf, table):
    def _z(r0, nq):
        for q in range(nq):
            pltpu.sync_copy(zbuf, table.at[pl.ds(r0 + q * 80, 80)])

    _row_split(s, lambda r0, n: _z(r0, 8), lambda r0, n: _z(r0, 5))


def _contrib_batch(F, ebuf, mb, degb, isdb, degS_t, rows_h, rows_s, crow):
    for k in range(_B // _L):
        ev = ebuf[pl.ds(k * _L, _L)]
        dv = plsc.load_gather(degS_t, [ev])
        degb[pl.ds(k * _L, _L)] = dv
        isdb[pl.ds(k * _L, _L)] = _rsqrt16(jnp.maximum(dv - 1.0, 1.0))

    def _j(j, _):
        jv = _splat(j)
        mj = plsc.load_gather(mb, [jv])
        dj = plsc.load_gather(degb, [jv])
        ij = plsc.load_gather(isdb, [jv])
        mask0 = (dj - mj) > 0.0
        for k in range(F // _L):
            sl = pl.ds(k * _L, _L)
            num = rows_s[j, sl] - rows_h[j, sl] * mj
            msk = mask0 & (num > 0.0)
            crow[j, sl] = jnp.where(msk, _sqrt16(num) * ij, 0.0)
        return 0
    lax.fori_loop(0, _B, _j, 0)


def _spow_stage(s, ept, hp_at, node_hbm, edge_hbm, nbufs, ebufs, rows,
                SpowS, sem_i, sem_g, sem_a):
    def _chunk(ci, _):
        ds = []
        for b in range(_K):
            b0 = s * ept + (ci * _K + b) * _B
            ds.append(pltpu.async_copy(node_hbm.at[pl.ds(b0, _B)],
                                       nbufs[b], sem_i))
            ds.append(pltpu.async_copy(edge_hbm.at[pl.ds(b0, _B)],
                                       ebufs[b], sem_i))
        _drain(ds)
        for b in range(_K):
            pltpu.sync_copy(hp_at(nbufs[b]), rows.at[b])
            pltpu.sync_copy(rows.at[b], SpowS.at[ebufs[b]], add=True)
        return 0
    lax.fori_loop(0, ept // (_B * _K), _chunk, 0)


def _contrib_stage(base, nbatch, F, hp_at, node_hbm, edge_hbm, mflt_hbm,
                   nbufs, ebufs, mbs, degb, isdb, degS_t, rows, crow1,
                   SpowS, OutS, sem_i, sem_g, sem_a):
    def _chunk(ci, _):
        ds = []
        for b in range(2):
            b0 = base + (ci * 2 + b) * _B
            ds.append(pltpu.async_copy(node_hbm.at[pl.ds(b0, _B)],
                                       nbufs[b], sem_i))
            ds.append(pltpu.async_copy(edge_hbm.at[pl.ds(b0, _B)],
                                       ebufs[b], sem_i))
            ds.append(pltpu.async_copy(mflt_hbm.at[pl.ds(b0, _B)],
                                       mbs[b], sem_i))
        _drain(ds)
        crows = [rows.at[4], crow1]
        for b in range(2):
            pltpu.sync_copy(hp_at(nbufs[b]), rows.at[b])
            pltpu.sync_copy(SpowS.at[ebufs[b]], rows.at[2 + b])
            _contrib_batch(F, ebufs[b], mbs[b], degb, isdb, degS_t,
                           rows.at[b], rows.at[2 + b], crows[b])
            pltpu.sync_copy(crows[b], OutS.at[nbufs[b]], add=True)
        return 0
    lax.fori_loop(0, nbatch // 2, _chunk, 0)


def _agg_refs(refs):
    nbufs = refs[0:_K]
    ebufs = refs[_K:2 * _K]
    return (nbufs, ebufs) + tuple(refs[2 * _K:])


def _agg1_body(hp_hbm, node_hbm, edge_hbm, mflt_hbm, degf_hbm,
               out_hbm, *refs):
    (nbufs, ebufs, mb0, mb1, degb, isdb, rows, crow1, degS_t, zbuf,
     sem_i, sem_g, sem_a, SpowS, OutS) = _agg_refs(refs)
    F = _D // _NC
    c = lax.axis_index("c")
    s = lax.axis_index("s")
    ept = _E // _NS

    def _zb(r, _):
        for k in range(F // _L):
            zbuf[r, pl.ds(k * _L, _L)] = jnp.zeros((_L,), jnp.float32)
        return 0
    lax.fori_loop(0, 80, _zb, 0)
    _zero_shared(s, zbuf, SpowS)
    _zero_shared(s, zbuf, OutS)
    pltpu.sync_copy(degf_hbm, degS_t)
    plsc.subcore_barrier()

    _spow_stage(s, ept, lambda ib: hp_hbm.at[c].at[ib], node_hbm, edge_hbm,
                nbufs, ebufs, rows, SpowS, sem_i, sem_g, sem_a)
    plsc.subcore_barrier()

    _contrib_stage(s * ept, ept // _B, F, lambda ib: hp_hbm.at[c].at[ib],
                   node_hbm, edge_hbm, mflt_hbm, nbufs, ebufs, [mb0, mb1],
                   degb, isdb, degS_t, rows, crow1, SpowS, OutS,
                   sem_i, sem_g, sem_a)
    plsc.subcore_barrier()

    _staged_copy(s, zbuf, lambda r0, n: out_hbm.at[c, pl.ds(r0, n)], OutS,
                 spm_to_hbm=True)


def _agg_scratch(F, shared):
    return ([pltpu.VMEM((_B,), jnp.int32)] * (2 * _K)
            + [pltpu.VMEM((_B,), jnp.float32)] * 4
            + [pltpu.VMEM((_K, _B, F), jnp.float32),
               pltpu.VMEM((_B, F), jnp.float32),
               pltpu.VMEM((_M,), jnp.float32),
               pltpu.VMEM((80, F), jnp.float32),
               pltpu.SemaphoreType.DMA,
               pltpu.SemaphoreType.DMA,
               pltpu.SemaphoreType.DMA]
            + shared)


_agg1 = pl.kernel(
    _agg1_body,
    out_type=jax.ShapeDtypeStruct((_NC, _N, _D // _NC), jnp.float32),
    mesh=plsc.VectorSubcoreMesh(**_MESH),
    compiler_params=_PARAMS,
    scratch_types=_agg_scratch(
        _D // _NC,
        [pltpu.VMEM_SHARED((_M, _D // _NC), jnp.float32),
         pltpu.VMEM_SHARED((_N, _D // _NC), jnp.float32)]),
)


def _agg2_body(hp_hbm, node_hbm, edge_hbm, mflt_hbm, degf_hbm,
               out_hbm, *refs):
    (nbufs, ebufs, mb0, mb1, degb, isdb, rows, crow1, degS_t, zbuf,
     sem_i, sem_g, sem_a, HpS, SpowS, OutS) = _agg_refs(refs)
    F = _HID
    c = lax.axis_index("c")
    s = lax.axis_index("s")

    _staged_copy(s, zbuf, lambda r0, n: hp_hbm.at[pl.ds(r0, n)], HpS,
                 spm_to_hbm=False)
    def _zb(r, _):
        zbuf[r, pl.ds(0, _L)] = jnp.zeros((_L,), jnp.float32)
        return 0
    lax.fori_loop(0, 80, _zb, 0)
    _zero_shared(s, zbuf, SpowS)
    _zero_shared(s, zbuf, OutS)
    pltpu.sync_copy(degf_hbm, degS_t)
    plsc.subcore_barrier()

    _spow_stage(s, _E // _NS, lambda ib: HpS.at[ib], node_hbm, edge_hbm,
                nbufs, ebufs, rows, SpowS, sem_i, sem_g, sem_a)
    plsc.subcore_barrier()

    ept_c = _E // (_NC * _NS)
    _contrib_stage((s * _NC + c) * ept_c, ept_c // _B, F,
                   lambda ib: HpS.at[ib], node_hbm, edge_hbm, mflt_hbm,
                   nbufs, ebufs, [mb0, mb1], degb, isdb, degS_t, rows,
                   crow1, SpowS, OutS, sem_i, sem_g, sem_a)
    plsc.subcore_barrier()

    _staged_copy(s, zbuf, lambda r0, n: out_hbm.at[c, pl.ds(r0, n)], OutS,
                 spm_to_hbm=True)


_agg2 = pl.kernel(
    _agg2_body,
    out_type=jax.ShapeDtypeStruct((_NC, _N, _HID), jnp.float32),
    mesh=plsc.VectorSubcoreMesh(**_MESH),
    compiler_params=_PARAMS,
    scratch_types=_agg_scratch(
        _HID,
        [pltpu.VMEM_SHARED((_N, _HID), jnp.float32),
         pltpu.VMEM_SHARED((_M, _HID), jnp.float32),
         pltpu.VMEM_SHARED((_N, _HID), jnp.float32)]),
)


def _prep1_body(x_ref, hc_ref, hp_ref):
    hc = jnp.clip(x_ref[...], 1e-07, 10.0)
    hc_ref[...] = hc
    hp = hc * hc
    hp_ref[0] = hp[:, :_D // _NC]
    hp_ref[1] = hp[:, _D // _NC:]


def _post1_body(hc_ref, os_ref, w_ref, b_ref, hc2_ref, hp2_ref):
    contrib = jnp.concatenate([os_ref[0], os_ref[1]], axis=1)
    ns = hc_ref[...] + contrib
    rowsum = ns.sum(axis=1)
    rinv = jnp.where(rowsum > 0.0, 1.0 / jnp.maximum(rowsum, 1e-30), 0.0)
    h = ns * rinv[:, None]
    h = jnp.dot(h, w_ref[...], preferred_element_type=jnp.float32) + b_ref[...]
    h = jnp.maximum(h, 0.0)
    hc2 = jnp.clip(h, 1e-07, 10.0)
    hc2_ref[...] = hc2
    hp2_ref[...] = hc2 * hc2


def _post2_body(hc_ref, os_ref, w_ref, b_ref, out_ref):
    ns = hc_ref[...] + os_ref[0] + os_ref[1]
    rowsum = ns.sum(axis=1)
    rinv = jnp.where(rowsum > 0.0, 1.0 / jnp.maximum(rowsum, 1e-30), 0.0)
    h = ns * rinv[:, None]
    out_ref[...] = (jnp.dot(h, w_ref[...], preferred_element_type=jnp.float32)
                    + b_ref[...])


def kernel(x, edge_index, W1, b1, W2, b2):
    node = edge_index[0].astype(jnp.int32)
    edge = edge_index[1].astype(jnp.int32)

    _t, mflt, degf = _stats(node, edge)

    hc1, hp1 = pl.pallas_call(
        _prep1_body,
        out_shape=[jax.ShapeDtypeStruct((_N, _D), jnp.float32),
                   jax.ShapeDtypeStruct((_NC, _N, _D // _NC), jnp.float32)],
    )(x)

    os1 = _agg1(hp1, node, edge, mflt, degf)

    hc2, hp2 = pl.pallas_call(
        _post1_body,
        out_shape=[jax.ShapeDtypeStruct((_N, _HID), jnp.float32),
                   jax.ShapeDtypeStruct((_N, _HID), jnp.float32)],
    )(hc1, os1, W1, b1)

    os2 = _agg2(hp2, node, edge, mflt, degf)

    out = pl.pallas_call(
        _post2_body,
        out_shape=jax.ShapeDtypeStruct((_N, _C), jnp.float32),
    )(hc2, os2, W2, b2)
    return out


# speedup vs baseline: 4.1161x; 1.3653x over previous
import jax
import jax.numpy as jnp
from jax import lax
from jax.experimental import pallas as pl
from jax.experimental.pallas import tpu as pltpu
from jax.experimental.pallas import tpu_sc as plsc

_N = 10000
_M = 10000
_E = 320000
_D = 128
_HID = 16
_C = 32
_NC = 2
_NS = 16
_L = 16
_B = 80
_K = 5

_MESH = dict(core_axis_name="c", subcore_axis_name="s", num_cores=_NC,
             num_subcores=_NS)
_PARAMS = pltpu.CompilerParams(needs_layout_passes=False,
                               use_tc_tiling_on_sc=False)


def _splat(v, dtype=jnp.int32):
    return jnp.full((_L,), v, dtype=dtype)


def _fill16(ref, off, val, dtype=jnp.float32):
    ref[pl.ds(off, _L)] = jnp.full((_L,), val, dtype=dtype)


def _rsqrt16(x):
    bi = plsc.bitcast(x, jnp.int32)
    y = plsc.bitcast(jnp.int32(0x5F3759DF) - (bi >> 1), jnp.float32)
    y = y * (1.5 - 0.5 * x * y * y)
    return y


def _sqrt16(x):
    return x * _rsqrt16(x)


def _row_split(s, fn640, fn400):
    @pl.when(s < _NS - 1)
    def _():
        fn640(s * 640, 640)

    @pl.when(s == _NS - 1)
    def _():
        fn400(9600, 400)


def _drain(ds):
    for d in ds:
        d.wait()


def _stats_body(node_hbm, edge_hbm, t_hbm, mflt_hbm, degf_hbm, *refs):
    nbufs = refs[0:_K]
    ebufs = refs[_K:2 * _K]
    pbufs = refs[2 * _K:3 * _K]
    vbufs = refs[3 * _K:4 * _K]
    wbufs = refs[4 * _K:5 * _K]
    fmbs = refs[5 * _K:6 * _K]
    (onesb, zbuf, p_all, w_all, sem_i, sem_g, sem_w, sem_a,
     degS, cS) = refs[6 * _K:]

    c = lax.axis_index("c")
    s = lax.axis_index("s")
    ept = _E // _NS
    nchunk = ept // (_B * _K)

    for k in range(_B // _L):
        _fill16(onesb, k * _L, 1.0)
    def _zb(i, _):
        _fill16(zbuf, i * _L, 0.0)
        return 0
    lax.fori_loop(0, 2000 // _L, _zb, 0)

    _row_split(s, lambda r0, n: pltpu.sync_copy(zbuf.at[pl.ds(0, 640)],
                                                degS.at[pl.ds(r0, 640)]),
               lambda r0, n: pltpu.sync_copy(zbuf.at[pl.ds(0, 400)],
                                             degS.at[pl.ds(r0, 400)]))
    def _zc(i, _):
        pltpu.sync_copy(zbuf.at[pl.ds(0, 2000)],
                        cS.at[pl.ds(s * ept + i * 2000, 2000)])
        return 0
    lax.fori_loop(0, ept // 2000, _zc, 0)
    plsc.subcore_barrier()

    @pl.when(c == 0)
    def _p1():
        def _chunk(ci, _):
            ds = []
            for b in range(_K):
                b0 = s * ept + (ci * _K + b) * _B
                ds.append(pltpu.async_copy(node_hbm.at[pl.ds(b0, _B)],
                                           nbufs[b], sem_i))
                ds.append(pltpu.async_copy(edge_hbm.at[pl.ds(b0, _B)],
                                           ebufs[b], sem_i))
            _drain(ds)
            for b in range(_K):
                boff = (ci * _K + b) * _B
                b0 = s * ept + boff
                for k in range(_B // _L):
                    sl = pl.ds(k * _L, _L)
                    pv = ebufs[b][sl] * _N + nbufs[b][sl]
                    pbufs[b][sl] = pv
                    p_all[pl.ds(boff + k * _L, _L)] = pv
                    vbufs[b][sl] = _splat(b0 + k * _L) + lax.iota(jnp.int32,
                                                                  _L)
                pltpu.sync_copy(vbufs[b], t_hbm.at[pbufs[b]])
                pltpu.sync_copy(onesb, degS.at[ebufs[b]], add=True)
            return 0
        lax.fori_loop(0, nchunk, _chunk, 0)
    plsc.subcore_barrier()

    @pl.when(c == 0)
    def _p2():
        def _batch(g, _):
            boff = g * _B
            pltpu.sync_copy(t_hbm.at[p_all.at[pl.ds(boff, _B)]], wbufs[0])
            pltpu.sync_copy(onesb, cS.at[wbufs[0]], add=True)
            for k in range(_B // _L):
                w_all[pl.ds(boff + k * _L, _L)] = wbufs[0][pl.ds(k * _L, _L)]
            return 0
        lax.fori_loop(0, ept // _B, _batch, 0)
    plsc.subcore_barrier()

    @pl.when(c == 0)
    def _p3():
        def _chunk(ci, _):
            ds = []
            for b in range(_K):
                boff = (ci * _K + b) * _B
                pltpu.sync_copy(cS.at[w_all.at[pl.ds(boff, _B)]], fmbs[b])
                ds.append(pltpu.async_copy(
                    fmbs[b], mflt_hbm.at[pl.ds(s * ept + boff, _B)], sem_w))
            _drain(ds)
            return 0
        lax.fori_loop(0, nchunk, _chunk, 0)

        def _wb(r0, n):
            pltpu.sync_copy(degS.at[pl.ds(r0, n)], zbuf.at[pl.ds(0, n)])
            pltpu.sync_copy(zbuf.at[pl.ds(0, n)], degf_hbm.at[pl.ds(r0, n)])
        _row_split(s, lambda r0, n: _wb(r0, 640), lambda r0, n: _wb(r0, 400))


_stats = pl.kernel(
    _stats_body,
    out_type=[jax.ShapeDtypeStruct((_N * _M,), jnp.int32),
              jax.ShapeDtypeStruct((_E,), jnp.float32),
              jax.ShapeDtypeStruct((_M,), jnp.float32)],
    mesh=plsc.VectorSubcoreMesh(**_MESH),
    compiler_params=_PARAMS,
    scratch_types=([pltpu.VMEM((_B,), jnp.int32)] * (5 * _K)
                   + [pltpu.VMEM((_B,), jnp.float32)] * _K
                   + [pltpu.VMEM((_B,), jnp.float32),
                      pltpu.VMEM((2000,), jnp.float32),
                      pltpu.VMEM((_E // _NS,), jnp.int32),
                      pltpu.VMEM((_E // _NS,), jnp.int32),
                      pltpu.SemaphoreType.DMA,
                      pltpu.SemaphoreType.DMA,
                      pltpu.SemaphoreType.DMA,
                      pltpu.SemaphoreType.DMA,
                      pltpu.VMEM_SHARED((_M,), jnp.float32),
                      pltpu.VMEM_SHARED((_E,), jnp.float32)]),
)


def _staged_copy(s, sbuf, mk_hbm_ref, spm, spm_to_hbm):
    def _chunk(r0, n):
        if spm_to_hbm:
            pltpu.sync_copy(spm.at[pl.ds(r0, n)], sbuf.at[pl.ds(0, n)])
            pltpu.sync_copy(sbuf.at[pl.ds(0, n)], mk_hbm_ref(r0, n))
        else:
            pltpu.sync_copy(mk_hbm_ref(r0, n), sbuf.at[pl.ds(0, n)])
            pltpu.sync_copy(sbuf.at[pl.ds(0, n)], spm.at[pl.ds(r0, n)])

    def _640(r0, _):
        for q in range(8):
            _chunk(r0 + q * 80, 80)

    def _400(r0, _):
        for q in range(5):
            _chunk(r0 + q * 80, 80)

    _row_split(s, _640, _400)


def _zero_shared(s, zbuf, table):
    def _z(r0, nq):
        for q in range(nq):
            pltpu.sync_copy(zbuf, table.at[pl.ds(r0 + q * 80, 80)])

    _row_split(s, lambda r0, n: _z(r0, 8), lambda r0, n: _z(r0, 5))


def _contrib_batch(F, ebuf, mb, degb, isdb, degS_t, rows_h, rows_s, crow):
    for k in range(_B // _L):
        ev = ebuf[pl.ds(k * _L, _L)]
        dv = plsc.load_gather(degS_t, [ev])
        degb[pl.ds(k * _L, _L)] = dv
        isdb[pl.ds(k * _L, _L)] = _rsqrt16(jnp.maximum(dv - 1.0, 1.0))

    def _j(j, _):
        jv = _splat(j)
        mj = plsc.load_gather(mb, [jv])
        dj = plsc.load_gather(degb, [jv])
        ij = plsc.load_gather(isdb, [jv])
        mask0 = (dj - mj) > 0.0
        for k in range(F // _L):
            sl = pl.ds(k * _L, _L)
            num = rows_s[j, sl] - rows_h[j, sl] * mj
            msk = mask0 & (num > 0.0)
            crow[j, sl] = jnp.where(msk, _sqrt16(num) * ij, 0.0)
        return 0
    lax.fori_loop(0, _B, _j, 0)


def _spow_stage(s, ept, hp_at, node_hbm, edge_hbm, nbufs, ebufs, rows,
                SpowS, sem_i, sem_g, sem_a):
    def _chunk(ci, _):
        ds = []
        for b in range(_K):
            b0 = s * ept + (ci * _K + b) * _B
            ds.append(pltpu.async_copy(node_hbm.at[pl.ds(b0, _B)],
                                       nbufs[b], sem_i))
            ds.append(pltpu.async_copy(edge_hbm.at[pl.ds(b0, _B)],
                                       ebufs[b], sem_i))
        _drain(ds)
        for b in range(_K):
            pltpu.sync_copy(hp_at(nbufs[b]), rows.at[b])
            pltpu.sync_copy(rows.at[b], SpowS.at[ebufs[b]], add=True)
        return 0
    lax.fori_loop(0, ept // (_B * _K), _chunk, 0)


def _contrib_stage(base, nbatch, F, hp_at, node_hbm, edge_hbm, mflt_hbm,
                   nbufs, ebufs, mbs, degb, isdb, degS_t, rows, crow1,
                   SpowS, OutS, sem_i, sem_g, sem_a):
    def _chunk(ci, _):
        ds = []
        for b in range(2):
            b0 = base + (ci * 2 + b) * _B
            ds.append(pltpu.async_copy(node_hbm.at[pl.ds(b0, _B)],
                                       nbufs[b], sem_i))
            ds.append(pltpu.async_copy(edge_hbm.at[pl.ds(b0, _B)],
                                       ebufs[b], sem_i))
            ds.append(pltpu.async_copy(mflt_hbm.at[pl.ds(b0, _B)],
                                       mbs[b], sem_i))
        _drain(ds)
        crows = [rows.at[4], crow1]
        for b in range(2):
            pltpu.sync_copy(hp_at(nbufs[b]), rows.at[b])
            pltpu.sync_copy(SpowS.at[ebufs[b]], rows.at[2 + b])
            _contrib_batch(F, ebufs[b], mbs[b], degb, isdb, degS_t,
                           rows.at[b], rows.at[2 + b], crows[b])
            pltpu.sync_copy(crows[b], OutS.at[nbufs[b]], add=True)
        return 0
    lax.fori_loop(0, nbatch // 2, _chunk, 0)


def _agg_refs(refs):
    nbufs = refs[0:_K]
    ebufs = refs[_K:2 * _K]
    return (nbufs, ebufs) + tuple(refs[2 * _K:])


def _agg1_body(hp_hbm, node_hbm, edge_hbm, mflt_hbm, degf_hbm,
               out_hbm, *refs):
    (nbufs, ebufs, mb0, mb1, degb, isdb, rows, crow1, degS_t, zbuf,
     sem_i, sem_g, sem_a, SpowS, OutS) = _agg_refs(refs)
    F = _D // _NC
    c = lax.axis_index("c")
    s = lax.axis_index("s")
    ept = _E // _NS

    def _zb(r, _):
        for k in range(F // _L):
            zbuf[r, pl.ds(k * _L, _L)] = jnp.zeros((_L,), jnp.float32)
        return 0
    lax.fori_loop(0, 80, _zb, 0)
    _zero_shared(s, zbuf, SpowS)
    _zero_shared(s, zbuf, OutS)
    pltpu.sync_copy(degf_hbm, degS_t)
    plsc.subcore_barrier()

    _spow_stage(s, ept, lambda ib: hp_hbm.at[c].at[ib], node_hbm, edge_hbm,
                nbufs, ebufs, rows, SpowS, sem_i, sem_g, sem_a)
    plsc.subcore_barrier()

    _contrib_stage(s * ept, ept // _B, F, lambda ib: hp_hbm.at[c].at[ib],
                   node_hbm, edge_hbm, mflt_hbm, nbufs, ebufs, [mb0, mb1],
                   degb, isdb, degS_t, rows, crow1, SpowS, OutS,
                   sem_i, sem_g, sem_a)
    plsc.subcore_barrier()

    _staged_copy(s, zbuf, lambda r0, n: out_hbm.at[c, pl.ds(r0, n)], OutS,
                 spm_to_hbm=True)


def _agg_scratch(F, shared):
    return ([pltpu.VMEM((_B,), jnp.int32)] * (2 * _K)
            + [pltpu.VMEM((_B,), jnp.float32)] * 4
            + [pltpu.VMEM((_K, _B, F), jnp.float32),
               pltpu.VMEM((_B, F), jnp.float32),
               pltpu.VMEM((_M,), jnp.float32),
               pltpu.VMEM((80, F), jnp.float32),
               pltpu.SemaphoreType.DMA,
               pltpu.SemaphoreType.DMA,
               pltpu.SemaphoreType.DMA]
            + shared)


_agg1 = pl.kernel(
    _agg1_body,
    out_type=jax.ShapeDtypeStruct((_NC, _N, _D // _NC), jnp.float32),
    mesh=plsc.VectorSubcoreMesh(**_MESH),
    compiler_params=_PARAMS,
    scratch_types=_agg_scratch(
        _D // _NC,
        [pltpu.VMEM_SHARED((_M, _D // _NC), jnp.float32),
         pltpu.VMEM_SHARED((_N, _D // _NC), jnp.float32)]),
)


def _agg2_body(hp_hbm, node_hbm, edge_hbm, mflt_hbm, degf_hbm,
               out_hbm, *refs):
    (nbufs, ebufs, mb0, mb1, degb, isdb, rows, crow1, degS_t, zbuf,
     sem_i, sem_g, sem_a, HpS, SpowS, OutS) = _agg_refs(refs)
    F = _HID
    c = lax.axis_index("c")
    s = lax.axis_index("s")

    _staged_copy(s, zbuf, lambda r0, n: hp_hbm.at[pl.ds(r0, n)], HpS,
                 spm_to_hbm=False)
    def _zb(r, _):
        zbuf[r, pl.ds(0, _L)] = jnp.zeros((_L,), jnp.float32)
        return 0
    lax.fori_loop(0, 80, _zb, 0)
    _zero_shared(s, zbuf, SpowS)
    _zero_shared(s, zbuf, OutS)
    pltpu.sync_copy(degf_hbm, degS_t)
    plsc.subcore_barrier()

    _spow_stage(s, _E // _NS, lambda ib: HpS.at[ib], node_hbm, edge_hbm,
                nbufs, ebufs, rows, SpowS, sem_i, sem_g, sem_a)
    plsc.subcore_barrier()

    ept_c = _E // (_NC * _NS)
    _contrib_stage((s * _NC + c) * ept_c, ept_c // _B, F,
                   lambda ib: HpS.at[ib], node_hbm, edge_hbm, mflt_hbm,
                   nbufs, ebufs, [mb0, mb1], degb, isdb, degS_t, rows,
                   crow1, SpowS, OutS, sem_i, sem_g, sem_a)
    plsc.subcore_barrier()

    _staged_copy(s, zbuf, lambda r0, n: out_hbm.at[c, pl.ds(r0, n)], OutS,
                 spm_to_hbm=True)


_agg2 = pl.kernel(
    _agg2_body,
    out_type=jax.ShapeDtypeStruct((_NC, _N, _HID), jnp.float32),
    mesh=plsc.VectorSubcoreMesh(**_MESH),
    compiler_params=_PARAMS,
    scratch_types=_agg_scratch(
        _HID,
        [pltpu.VMEM_SHARED((_N, _HID), jnp.float32),
         pltpu.VMEM_SHARED((_M, _HID), jnp.float32),
         pltpu.VMEM_SHARED((_N, _HID), jnp.float32)]),
)


def _prep1_body(x_ref, hc_ref, hp_ref):
    hc = jnp.clip(x_ref[...], 1e-07, 10.0)
    hc_ref[...] = hc
    hp = hc * hc
    hp_ref[0] = hp[:, :_D // _NC]
    hp_ref[1] = hp[:, _D // _NC:]


def _post1_body(hc_ref, os_ref, w_ref, b_ref, hc2_ref, hp2_ref):
    contrib = jnp.concatenate([os_ref[0], os_ref[1]], axis=1)
    ns = hc_ref[...] + contrib
    rowsum = ns.sum(axis=1)
    rinv = jnp.where(rowsum > 0.0, 1.0 / jnp.maximum(rowsum, 1e-30), 0.0)
    h = ns * rinv[:, None]
    h = jnp.dot(h, w_ref[...], preferred_element_type=jnp.float32) + b_ref[...]
    h = jnp.maximum(h, 0.0)
    hc2 = jnp.clip(h, 1e-07, 10.0)
    hc2_ref[...] = hc2
    hp2_ref[...] = hc2 * hc2


def _post2_body(hc_ref, os_ref, w_ref, b_ref, out_ref):
    ns = hc_ref[...] + os_ref[0] + os_ref[1]
    rowsum = ns.sum(axis=1)
    rinv = jnp.where(rowsum > 0.0, 1.0 / jnp.maximum(rowsum, 1e-30), 0.0)
    h = ns * rinv[:, None]
    out_ref[...] = (jnp.dot(h, w_ref[...], preferred_element_type=jnp.float32)
                    + b_ref[...])


def kernel(x, edge_index, W1, b1, W2, b2):
    node = edge_index[0].astype(jnp.int32)
    edge = edge_index[1].astype(jnp.int32)

    _t, mflt, degf = _stats(node, edge)

    hc1, hp1 = pl.pallas_call(
        _prep1_body,
        out_shape=[jax.ShapeDtypeStruct((_N, _D), jnp.float32),
                   jax.ShapeDtypeStruct((_NC, _N, _D // _NC), jnp.float32)],
    )(x)

    os1 = _agg1(hp1, node, edge, mflt, degf)

    hc2, hp2 = pl.pallas_call(
        _post1_body,
        out_shape=[jax.ShapeDtypeStruct((_N, _HID), jnp.float32),
                   jax.ShapeDtypeStruct((_N, _HID), jnp.float32)],
    )(hc1, os1, W1, b1)

    os2 = _agg2(hp2, node, edge, mflt, degf)

    out = pl.pallas_call(
        _post2_body,
        out_shape=jax.ShapeDtypeStruct((_N, _C), jnp.float32),
    )(hc2, os2, W2, b2)
    return out
